# Initial kernel scaffold; baseline (speedup 1.0000x reference)
import jax
import jax.numpy as jnp
from jax.experimental import pallas as pl


def kernel(x, edge_index, edge_attr, Wv, Wo, lq1, lk1, W1, b1, W2, b2):
    raise NotImplementedError("write your pallas kernel here")


# speedup vs baseline: 27.9473x; 27.9473x over previous
import functools
import math

import jax
import jax.numpy as jnp
from jax import lax
from jax.experimental import pallas as pl
from jax.experimental.pallas import tpu as pltpu
from jax.experimental.pallas import tpu_sc as plsc

NC = 2
NS = 16
NW = NC * NS
C = 128


def _vproj_body(x_ref, wv_ref, v_ref):
    v_ref[...] = lax.dot_general(
        x_ref[...], wv_ref[...], (((1,), (1,)), ((), ())),
        preferred_element_type=jnp.float32)


def _vproj(x, Wv):
    N, D = x.shape
    blk = 2000
    return pl.pallas_call(
        _vproj_body,
        grid=(N // blk,),
        in_specs=[pl.BlockSpec((blk, D), lambda i: (i, 0)),
                  pl.BlockSpec((D, D), lambda i: (0, 0))],
        out_specs=pl.BlockSpec((blk, D), lambda i: (i, 0)),
        out_shape=jax.ShapeDtypeStruct((N, D), jnp.float32),
    )(x, Wv)


def _encoder_body(ea_ref, w1_ref, b1_ref, w2_ref, b2_ref, s_ref):
    h = lax.dot_general(ea_ref[...], w1_ref[...], (((1,), (1,)), ((), ())),
                        preferred_element_type=jnp.float32) + b1_ref[...]
    h = 0.5 * h * (1.0 + lax.erf(h * (1.0 / math.sqrt(2.0))))
    s_ref[...] = lax.dot_general(h, w2_ref[...], (((1,), (0,)), ((), ())),
                                 preferred_element_type=jnp.float32) + b2_ref[...]


def _encoder(edge_attr, W1, b1, W2p_t, b2p):
    E, ED = edge_attr.shape
    D = W1.shape[0]
    blk = 4000
    return pl.pallas_call(
        _encoder_body,
        grid=(E // blk,),
        in_specs=[pl.BlockSpec((blk, ED), lambda i: (i, 0)),
                  pl.BlockSpec((D, ED), lambda i: (0, 0)),
                  pl.BlockSpec((1, D), lambda i: (0, 0)),
                  pl.BlockSpec((D, 16), lambda i: (0, 0)),
                  pl.BlockSpec((1, 16), lambda i: (0, 0))],
        out_specs=pl.BlockSpec((blk, 16), lambda i: (i, 0)),
        out_shape=jax.ShapeDtypeStruct((E, 16), jnp.float32),
    )(edge_attr, W1, b1.reshape(1, D), W2p_t, b2p.reshape(1, 16))


def _oproj_body(p_ref, wo_ref, o_ref):
    p = p_ref[0] + p_ref[1]
    o_ref[...] = lax.dot_general(
        p, wo_ref[...], (((1,), (1,)), ((), ())),
        preferred_element_type=jnp.float32)


def _oproj(parts, Wo):
    _, N, D = parts.shape
    blk = 2000
    return pl.pallas_call(
        _oproj_body,
        grid=(N // blk,),
        in_specs=[pl.BlockSpec((2, blk, D), lambda i: (0, i, 0)),
                  pl.BlockSpec((D, D), lambda i: (0, 0))],
        out_specs=pl.BlockSpec((blk, D), lambda i: (i, 0)),
        out_shape=jax.ShapeDtypeStruct((N, D), jnp.float32),
    )(parts, Wo)


def _sc_denoms(S3, tgt2, m16, zden):
    NCH = S3.shape[0]
    N = zden.shape[0]
    kmax = -(-NCH // NW)
    rows = N // NS
    mesh = plsc.VectorSubcoreMesh(core_axis_name="c", subcore_axis_name="s",
                                  num_cores=NC, num_subcores=NS)

    @functools.partial(
        pl.kernel,
        out_type=jax.ShapeDtypeStruct((NC, N, 16), jnp.float32),
        mesh=mesh,
        compiler_params=pltpu.CompilerParams(use_tc_tiling_on_sc=False),
        scratch_types=[
            pltpu.VMEM((C, 16), jnp.float32),
            pltpu.VMEM((1, C), jnp.int32),
            pltpu.VMEM((16,), jnp.float32),
            pltpu.VMEM_SHARED((N, 16), jnp.float32),
        ],
    )
    def den_kernel(s3, tgt2_r, m16_r, zden_r, den_out, s_v, t_v, m_v, den_sh):
        cc = lax.axis_index("c")
        ss = lax.axis_index("s")
        w = ss * NC + cc
        pltpu.sync_copy(zden_r.at[pl.ds(ss * rows, rows)],
                        den_sh.at[pl.ds(ss * rows, rows)])
        pltpu.sync_copy(m16_r, m_v)
        plsc.subcore_barrier()
        mv = m_v[...]

        def chunk_body(k, carry):
            cid = w + NW * k

            @pl.when(cid < NCH)
            def _():
                pltpu.sync_copy(tgt2_r.at[cid], t_v)
                pltpu.sync_copy(s3.at[cid], s_v)

                def inner(i, c2):
                    s_v[i] = jnp.exp(s_v[i] - mv)
                    return c2
                lax.fori_loop(0, C, inner, 0)
                pltpu.sync_copy(s_v, den_sh.at[t_v.at[0]], add=True)
            return carry

        lax.fori_loop(0, kmax, chunk_body, 0)
        plsc.subcore_barrier()
        pltpu.sync_copy(den_sh.at[pl.ds(ss * rows, rows)],
                        den_out.at[cc, pl.ds(ss * rows, rows)])

    return den_kernel(S3, tgt2, m16, zden)


def _sc_aggregate(S3, tgt2, src2, V, den0, den1, m16, scale16, zacc):
    NCH = S3.shape[0]
    N, D = zacc.shape
    kmax = -(-NCH // NW)
    rows = N // NS
    mesh = plsc.VectorSubcoreMesh(core_axis_name="c", subcore_axis_name="s",
                                  num_cores=NC, num_subcores=NS)

    @functools.partial(
        pl.kernel,
        out_type=(jax.ShapeDtypeStruct((NCH, C, 16), jnp.float32),
                  jax.ShapeDtypeStruct((NC, N, D), jnp.float32)),
        mesh=mesh,
        compiler_params=pltpu.CompilerParams(use_tc_tiling_on_sc=False),
        scratch_types=[
            pltpu.VMEM((C, 16), jnp.float32),
            pltpu.VMEM((1, C), jnp.int32),
            pltpu.VMEM((1, C), jnp.int32),
            pltpu.VMEM((C, 16), jnp.float32),
            pltpu.VMEM((C, 16), jnp.float32),
            pltpu.VMEM((C, 16), jnp.float32),
            pltpu.VMEM((C, D), jnp.float32),
            pltpu.VMEM((24,), jnp.float32),
            pltpu.VMEM((16,), jnp.float32),
            pltpu.VMEM((16,), jnp.float32),
            pltpu.VMEM_SHARED((N, D), jnp.float32),
        ],
    )
    def agg_kernel(s3, tgt2_r, src2_r, v_r, den0_r, den1_r, m16_r, scale_r,
                   zacc_r, attn_out, parts_out,
                   s_v, t_v, r_v, d0_v, d1_v, a_v, vr_v, q_b, m_v, sc_v,
                   acc_sh):
        cc = lax.axis_index("c")
        ss = lax.axis_index("s")
        w = ss * NC + cc
        pltpu.sync_copy(zacc_r.at[pl.ds(ss * rows, rows)],
                        acc_sh.at[pl.ds(ss * rows, rows)])
        pltpu.sync_copy(m16_r, m_v)
        pltpu.sync_copy(scale_r, sc_v)
        plsc.subcore_barrier()
        mv = m_v[...]
        scv = sc_v[...]

        def chunk_body(k, carry):
            cid = w + NW * k

            @pl.when(cid < NCH)
            def _():
                pltpu.sync_copy(tgt2_r.at[cid], t_v)
                pltpu.sync_copy(src2_r.at[cid], r_v)
                pltpu.sync_copy(s3.at[cid], s_v)
                pltpu.sync_copy(den0_r.at[t_v.at[0]], d0_v)
                pltpu.sync_copy(den1_r.at[t_v.at[0]], d1_v)
                pltpu.sync_copy(v_r.at[r_v.at[0]], vr_v)

                def inner(i, c2):
                    p = jnp.exp(s_v[i] - mv)
                    d = (d0_v[i] + d1_v[i] + 1e-16) * scv
                    q = p / d
                    q_b[pl.ds(0, 16)] = q
                    u = q_b[pl.ds(8, 16)]
                    a = q - u
                    a_v[i] = a
                    for j in range(8):
                        wj = a[j]
                        col = 16 * j
                        vr_v[i, pl.ds(col, 16)] = vr_v[i, pl.ds(col, 16)] * wj
                    return c2
                lax.fori_loop(0, C, inner, 0)
                pltpu.sync_copy(a_v, attn_out.at[cid])
                pltpu.sync_copy(vr_v, acc_sh.at[t_v.at[0]], add=True)
            return carry

        lax.fori_loop(0, kmax, chunk_body, 0)
        plsc.subcore_barrier()
        pltpu.sync_copy(acc_sh.at[pl.ds(ss * rows, rows)],
                        parts_out.at[cc, pl.ds(ss * rows, rows)])

    return agg_kernel(S3, tgt2, src2, V, den0, den1, m16, scale16, zacc)


def kernel(x, edge_index, edge_attr, Wv, Wo, lq1, lk1, W1, b1, W2, b2):
    N, D = x.shape
    E = edge_attr.shape[0]
    H = 8

    lambda_init = 1.0 - math.exp(-1.0)
    lambda_1 = jnp.exp(jnp.sum(lq1 * lk1))
    beta = jax.nn.sigmoid(lambda_1 * lambda_init)

    src = edge_index[0].astype(jnp.int32)
    tgt = edge_index[1].astype(jnp.int32)

    perm = jnp.array([0, 2, 4, 6, 8, 10, 12, 14, 1, 3, 5, 7, 9, 11, 13, 15])
    W2p_t = W2[perm].T
    b2p = b2[perm]

    V = _vproj(x, Wv)
    S = _encoder(edge_attr, W1, b1, W2p_t, b2p)
    m = jnp.max(S)
    m16 = jnp.full((16,), m, jnp.float32)
    scale16 = jnp.where(jnp.arange(16) < 8, 1.0, 1.0 / beta).astype(jnp.float32)

    NCH = E // C
    Np = ((N + 127) // 128) * 128
    S3 = S.reshape(NCH, C, 16)
    tgt3 = tgt.reshape(NCH, 1, C)
    src3 = src.reshape(NCH, 1, C)
    zden = jnp.zeros((Np, 16), jnp.float32)
    zacc = jnp.zeros((Np, D), jnp.float32)

    den = _sc_denoms(S3, tgt3, m16, zden)
    attn16, parts = _sc_aggregate(S3, tgt3, src3, V, den[0], den[1],
                                  m16, scale16, zacc)
    out = _oproj(parts, Wo)[:N]
    attn = attn16.reshape(E, 16)[:, :H]
    return out, attn, beta
